# Initial kernel scaffold; baseline (speedup 1.0000x reference)
import jax
import jax.numpy as jnp
from jax.experimental import pallas as pl


def kernel(initial_position, contact_node, parent2child, branch, contact_force, edge_index, batch, params):
    raise NotImplementedError("write your pallas kernel here")


# speedup vs baseline: 3.1591x; 3.1591x over previous
import functools

import jax
import jax.numpy as jnp
from jax import lax
from jax.experimental import pallas as pl
from jax.experimental.pallas import tpu as pltpu
from jax.experimental.pallas import tpu_sc as plsc

F32 = jnp.float32

NC = 2
NS = 16
NW = NC * NS

EBLK = 2000
NBLK = 2000
C = 128


def _dot(a, b):
    return lax.dot_general(a, b, (((1,), (0,)), ((), ())),
                           preferred_element_type=F32)


def _sc_gather_pair(A, B, dst, src):
    E, H = A.shape[0] and dst.shape[0], A.shape[1]
    E = dst.shape[0]
    nchunk = E // C
    mesh = plsc.VectorSubcoreMesh(core_axis_name="c", subcore_axis_name="s")

    @functools.partial(
        pl.kernel, mesh=mesh,
        out_type=(jax.ShapeDtypeStruct((E, H), F32),
                  jax.ShapeDtypeStruct((E, H), F32)),
        scratch_types=[pltpu.VMEM((C,), jnp.int32),
                       pltpu.VMEM((C,), jnp.int32),
                       pltpu.VMEM((C, H), F32),
                       pltpu.VMEM((C, H), F32),
                       pltpu.SemaphoreType.DMA,
                       pltpu.SemaphoreType.DMA],
    )
    def k(a_hbm, b_hbm, dst_hbm, src_hbm, ga_hbm, gb_hbm,
          idx_d, idx_s, buf_a, buf_b, sem_a, sem_b):
        wid = lax.axis_index("s") * NC + lax.axis_index("c")
        nj = (nchunk - wid + NW - 1) // NW

        def body(j, carry):
            jj = wid + j * NW
            off = jj * C
            pltpu.sync_copy(dst_hbm.at[pl.ds(off, C)], idx_d)
            pltpu.sync_copy(src_hbm.at[pl.ds(off, C)], idx_s)
            cpa = pltpu.async_copy(a_hbm.at[idx_d], buf_a, sem_a)
            cpb = pltpu.async_copy(b_hbm.at[idx_s], buf_b, sem_b)
            cpa.wait()
            cpb.wait()
            pltpu.sync_copy(buf_a, ga_hbm.at[pl.ds(off, C)])
            pltpu.sync_copy(buf_b, gb_hbm.at[pl.ds(off, C)])
            return carry

        lax.fori_loop(0, nj, body, 0)

    return k(A, B, dst, src)


def _sc_gather_batch_src(batch, src):
    E = src.shape[0]
    nchunk = E // C
    mesh = plsc.VectorSubcoreMesh(core_axis_name="c", subcore_axis_name="s")

    @functools.partial(
        pl.kernel, mesh=mesh,
        out_type=jax.ShapeDtypeStruct((E,), jnp.int32),
        scratch_types=[pltpu.VMEM((C,), jnp.int32),
                       pltpu.VMEM((C,), jnp.int32),
                       pltpu.SemaphoreType.DMA],
    )
    def k(batch_hbm, src_hbm, out_hbm, idx_v, buf, sem):
        wid = lax.axis_index("s") * NC + lax.axis_index("c")
        nj = (nchunk - wid + NW - 1) // NW

        def body(j, carry):
            jj = wid + j * NW
            off = jj * C
            pltpu.sync_copy(src_hbm.at[pl.ds(off, C)], idx_v)
            pltpu.async_copy(batch_hbm.at[idx_v], buf, sem).wait()
            pltpu.sync_copy(buf, out_hbm.at[pl.ds(off, C)])
            return carry

        lax.fori_loop(0, nj, body, 0)

    return k(batch, src)


def _sc_scatter_add(msg, dst, zeros_nh):
    E, H = msg.shape
    N = zeros_nh.shape[0]
    nchunk = E // C
    ZCH = 40
    nzc = N // ZCH
    mesh = plsc.VectorSubcoreMesh(core_axis_name="c", subcore_axis_name="s")

    @functools.partial(
        pl.kernel, mesh=mesh,
        out_type=jax.ShapeDtypeStruct((NC, N, H), F32),
        scratch_types=[pltpu.VMEM((C,), jnp.int32),
                       pltpu.VMEM((C, H), F32),
                       pltpu.VMEM_SHARED((N, H), F32)],
    )
    def k(msg_hbm, dst_hbm, zero_hbm, out_hbm, idx_v, buf, acc):
        cid = lax.axis_index("c")
        sid = lax.axis_index("s")
        wid = sid * NC + cid
        nz = (nzc - sid + NS - 1) // NS

        def zbody(j, carry):
            r0 = (sid + j * NS) * ZCH
            pltpu.sync_copy(zero_hbm.at[pl.ds(r0, ZCH)],
                            acc.at[pl.ds(r0, ZCH)])
            return carry

        lax.fori_loop(0, nz, zbody, 0)
        plsc.subcore_barrier()

        nj = (nchunk - wid + NW - 1) // NW

        def body(j, carry):
            jj = wid + j * NW
            off = jj * C
            pltpu.sync_copy(dst_hbm.at[pl.ds(off, C)], idx_v)
            pltpu.sync_copy(msg_hbm.at[pl.ds(off, C)], buf)
            pltpu.sync_copy(buf, acc.at[idx_v], add=True)
            return carry

        lax.fori_loop(0, nj, body, 0)
        plsc.subcore_barrier()

        def wbody(j, carry):
            r0 = (sid + j * NS) * ZCH
            pltpu.sync_copy(acc.at[pl.ds(r0, ZCH)],
                            out_hbm.at[cid, pl.ds(r0, ZCH)])
            return carry

        lax.fori_loop(0, nz, wbody, 0)

    return k(msg, dst, zeros_nh)


def _tc_counts(batch_src_2d, g):
    E = batch_src_2d.shape[0]
    grid = (E // EBLK,)

    def body(bs_ref, out_ref):
        i = pl.program_id(0)

        @pl.when(i == 0)
        def _():
            out_ref[...] = jnp.zeros_like(out_ref)

        oh = (bs_ref[...] == lax.broadcasted_iota(jnp.int32, (EBLK, g), 1))
        out_ref[...] += jnp.sum(oh.astype(F32), axis=0, keepdims=True)

    return pl.pallas_call(
        body,
        grid=grid,
        in_specs=[pl.BlockSpec((EBLK, 1), lambda i: (i, 0))],
        out_specs=pl.BlockSpec((1, g), lambda i: (0, 0)),
        out_shape=jax.ShapeDtypeStruct((1, g), F32),
    )(batch_src_2d)


def _mlp3(h, w1, b1, w2, b2, w3, b3):
    h = jax.nn.relu(_dot(h, w1) + b1)
    h = jax.nn.relu(_dot(h, w2) + b2)
    return _dot(h, w3) + b3


def _tc_node_encode_prep(x4, enc, w1a_b1, w1b):
    N = x4.shape[0]
    H = enc[0][0].shape[1]
    grid = (N // NBLK,)
    (we1, be1), (we2, be2), (we3, be3) = enc
    w1a, b1 = w1a_b1

    def body(x4_ref, we1_r, be1_r, we2_r, be2_r, we3_r, be3_r,
             w1a_r, b1_r, w1b_r, x0_ref, a_ref, b_ref):
        x0 = _mlp3(x4_ref[...], we1_r[...], be1_r[...], we2_r[...],
                   be2_r[...], we3_r[...], be3_r[...])
        x0_ref[...] = x0
        a_ref[...] = _dot(x0, w1a_r[...]) + b1_r[...]
        b_ref[...] = _dot(x0, w1b_r[...])

    full = lambda a: pl.BlockSpec(a.shape, lambda i: (0,) * a.ndim)
    return pl.pallas_call(
        body,
        grid=grid,
        in_specs=[pl.BlockSpec((NBLK, 4), lambda i: (i, 0)),
                  full(we1), full(be1), full(we2), full(be2),
                  full(we3), full(be3), full(w1a), full(b1), full(w1b)],
        out_specs=[pl.BlockSpec((NBLK, H), lambda i: (i, 0))] * 3,
        out_shape=[jax.ShapeDtypeStruct((N, H), F32)] * 3,
    )(x4, we1, be1, we2, be2, we3, be3, w1a, b1, w1b)


def _edge_core(ga, gb, ef, e0, counts, gf, w1c, w1d, w2, b2, w3, b3, eblk, g):
    lt = (lax.broadcasted_iota(jnp.int32, (g, g), 0)
          < lax.broadcasted_iota(jnp.int32, (g, g), 1)).astype(F32)
    starts = _dot(counts, lt).astype(jnp.int32)
    ends = starts + counts.astype(jnp.int32)
    ridx = lax.broadcasted_iota(jnp.int32, (eblk, g), 0) + e0
    oh = (ridx >= starts).astype(F32) - (ridx >= ends).astype(F32)
    gf_tab = _dot(gf, w1d)
    h1 = jax.nn.relu(ga + gb + _dot(ef, w1c) + _dot(oh, gf_tab))
    h2 = jax.nn.relu(_dot(h1, w2) + b2)
    return _dot(h2, w3) + b3


def _tc_edge_encode_msg(attr, ga, gb, counts, gf, enc, esplit):
    E = attr.shape[0]
    H = ga.shape[1]
    g = gf.shape[0]
    grid = (E // EBLK,)
    (we1, be1), (we2, be2), (we3, be3) = enc
    w1c, w1d, w2, b2, w3, b3 = esplit

    def body(attr_ref, ga_ref, gb_ref, counts_r, gf_r,
             we1_r, be1_r, we2_r, be2_r, we3_r, be3_r,
             w1c_r, w1d_r, w2_r, b2_r, w3_r, b3_r,
             msg_ref, ef_ref):
        ef = _mlp3(attr_ref[...], we1_r[...], be1_r[...], we2_r[...],
                   be2_r[...], we3_r[...], be3_r[...])
        e0 = pl.program_id(0) * EBLK
        msg = _edge_core(ga_ref[...], gb_ref[...], ef, e0, counts_r[...],
                         gf_r[...], w1c_r[...], w1d_r[...], w2_r[...],
                         b2_r[...], w3_r[...], b3_r[...], EBLK, g)
        msg_ref[...] = msg
        ef_ref[...] = ef + msg

    full = lambda a: pl.BlockSpec(a.shape, lambda i: (0,) * a.ndim)
    return pl.pallas_call(
        body,
        grid=grid,
        in_specs=[pl.BlockSpec((EBLK, 2), lambda i: (i, 0)),
                  pl.BlockSpec((EBLK, H), lambda i: (i, 0)),
                  pl.BlockSpec((EBLK, H), lambda i: (i, 0)),
                  full(counts), full(gf),
                  full(we1), full(be1), full(we2), full(be2),
                  full(we3), full(be3),
                  full(w1c), full(w1d), full(w2), full(b2),
                  full(w3), full(b3)],
        out_specs=[pl.BlockSpec((EBLK, H), lambda i: (i, 0))] * 2,
        out_shape=[jax.ShapeDtypeStruct((E, H), F32)] * 2,
    )(attr, ga, gb, counts, gf, we1, be1, we2, be2, we3, be3,
      w1c, w1d, w2, b2, w3, b3)


def _tc_edge_msg(ef_in, ga, gb, counts, gf, esplit):
    E, H = ef_in.shape
    g = gf.shape[0]
    grid = (E // EBLK,)
    w1c, w1d, w2, b2, w3, b3 = esplit

    def body(ef_in_ref, ga_ref, gb_ref, counts_r, gf_r,
             w1c_r, w1d_r, w2_r, b2_r, w3_r, b3_r, msg_ref, ef_ref):
        ef = ef_in_ref[...]
        e0 = pl.program_id(0) * EBLK
        msg = _edge_core(ga_ref[...], gb_ref[...], ef, e0, counts_r[...],
                         gf_r[...], w1c_r[...], w1d_r[...], w2_r[...],
                         b2_r[...], w3_r[...], b3_r[...], EBLK, g)
        msg_ref[...] = msg
        ef_ref[...] = ef + msg

    full = lambda a: pl.BlockSpec(a.shape, lambda i: (0,) * a.ndim)
    return pl.pallas_call(
        body,
        grid=grid,
        in_specs=[pl.BlockSpec((EBLK, H), lambda i: (i, 0))] * 3
                 + [full(counts), full(gf), full(w1c), full(w1d),
                    full(w2), full(b2), full(w3), full(b3)],
        out_specs=[pl.BlockSpec((EBLK, H), lambda i: (i, 0))] * 2,
        out_shape=[jax.ShapeDtypeStruct((E, H), F32)] * 2,
    )(ef_in, ga, gb, counts, gf, w1c, w1d, w2, b2, w3, b3)


def _node_update(x, p_ref0, p_ref1, batch_blk, gf, u1a, u1b, u1c, c1,
                 u2, c2, u3, c3, g):
    aggr = p_ref0 + p_ref1
    oh = (batch_blk == lax.broadcasted_iota(jnp.int32, (batch_blk.shape[0], g), 1)).astype(F32)
    gfn_tab = _dot(gf, u1c)
    h1 = jax.nn.relu(_dot(x, u1a) + _dot(aggr, u1b) + _dot(oh, gfn_tab) + c1)
    h2 = jax.nn.relu(_dot(h1, u2) + c2)
    return x + _dot(h2, u3) + c3


def _tc_node_update_prep(x, partials, batch2d, gf, nsplit, w1a_b1, w1b):
    N, H = x.shape
    g = gf.shape[0]
    grid = (N // NBLK,)
    u1a, u1b, u1c, c1, u2, c2, u3, c3 = nsplit
    w1a, b1 = w1a_b1

    def body(x_ref, p_ref, b_ref_in, gf_r, u1a_r, u1b_r, u1c_r, c1_r,
             u2_r, c2_r, u3_r, c3_r, w1a_r, b1_r, w1b_r,
             xn_ref, a_ref, b_ref):
        xn = _node_update(x_ref[...], p_ref[0], p_ref[1], b_ref_in[...],
                          gf_r[...], u1a_r[...], u1b_r[...], u1c_r[...],
                          c1_r[...], u2_r[...], c2_r[...], u3_r[...],
                          c3_r[...], g)
        xn_ref[...] = xn
        a_ref[...] = _dot(xn, w1a_r[...]) + b1_r[...]
        b_ref[...] = _dot(xn, w1b_r[...])

    full = lambda a: pl.BlockSpec(a.shape, lambda i: (0,) * a.ndim)
    return pl.pallas_call(
        body,
        grid=grid,
        in_specs=[pl.BlockSpec((NBLK, H), lambda i: (i, 0)),
                  pl.BlockSpec((NC, NBLK, H), lambda i: (0, i, 0)),
                  pl.BlockSpec((NBLK, 1), lambda i: (i, 0)),
                  full(gf), full(u1a), full(u1b), full(u1c), full(c1),
                  full(u2), full(c2), full(u3), full(c3),
                  full(w1a), full(b1), full(w1b)],
        out_specs=[pl.BlockSpec((NBLK, H), lambda i: (i, 0))] * 3,
        out_shape=[jax.ShapeDtypeStruct((N, H), F32)] * 3,
    )(x, partials, batch2d, gf, u1a, u1b, u1c, c1, u2, c2, u3, c3,
      w1a, b1, w1b)


def _tc_node_update_decode(x, partials, batch2d, gf, nsplit, dec):
    N, H = x.shape
    g = gf.shape[0]
    grid = (N // NBLK,)
    u1a, u1b, u1c, c1, u2, c2, u3, c3 = nsplit
    (wd1, bd1), (wd2, bd2), (wd3, bd3) = dec
    O = wd3.shape[1]

    def body(x_ref, p_ref, b_ref_in, gf_r, u1a_r, u1b_r, u1c_r, c1_r,
             u2_r, c2_r, u3_r, c3_r, wd1_r, bd1_r, wd2_r, bd2_r,
             wd3_r, bd3_r, out_ref):
        xn = _node_update(x_ref[...], p_ref[0], p_ref[1], b_ref_in[...],
                          gf_r[...], u1a_r[...], u1b_r[...], u1c_r[...],
                          c1_r[...], u2_r[...], c2_r[...], u3_r[...],
                          c3_r[...], g)
        out_ref[...] = _mlp3(xn, wd1_r[...], bd1_r[...], wd2_r[...],
                             bd2_r[...], wd3_r[...], bd3_r[...])

    full = lambda a: pl.BlockSpec(a.shape, lambda i: (0,) * a.ndim)
    return pl.pallas_call(
        body,
        grid=grid,
        in_specs=[pl.BlockSpec((NBLK, H), lambda i: (i, 0)),
                  pl.BlockSpec((NC, NBLK, H), lambda i: (0, i, 0)),
                  pl.BlockSpec((NBLK, 1), lambda i: (i, 0)),
                  full(gf), full(u1a), full(u1b), full(u1c), full(c1),
                  full(u2), full(c2), full(u3), full(c3),
                  full(wd1), full(bd1), full(wd2), full(bd2),
                  full(wd3), full(bd3)],
        out_specs=pl.BlockSpec((NBLK, O), lambda i: (i, 0)),
        out_shape=jax.ShapeDtypeStruct((N, O), F32),
    )(x, partials, batch2d, gf, u1a, u1b, u1c, c1, u2, c2, u3, c3,
      wd1, bd1, wd2, bd2, wd3, bd3)


def _split_edge_params(lin_edge, h):
    (w1, b1), (w2, b2), (w3, b3) = lin_edge
    w1a = w1[:h]
    w1b = w1[h:2 * h]
    w1c = w1[2 * h:3 * h]
    w1d = w1[3 * h:]
    r = lambda b: b.reshape(1, -1)
    return (w1a, r(b1)), w1b, (w1c, w1d, w2, r(b2), w3, r(b3))


def _split_node_params(lin_node, h):
    (u1, c1), (u2, c2), (u3, c3) = lin_node
    r = lambda b: b.reshape(1, -1)
    return (u1[:h], u1[h:2 * h], u1[2 * h:], r(c1), u2, r(c2), u3, r(c3))


def kernel(initial_position, contact_node, parent2child, branch,
           contact_force, edge_index, batch, params):
    N = initial_position.shape[0]
    E = parent2child.shape[0]
    H = params["node_in"][-1][0].shape[1]
    gf = contact_force.reshape(-1, 3)
    g = gf.shape[0]

    src = edge_index[0]
    dst = edge_index[1]
    x4 = jnp.concatenate([initial_position, contact_node[:, None]], axis=-1)
    attr = jnp.stack([parent2child, branch], axis=-1)
    batch2d = batch[:, None]
    zeros_nh = jnp.zeros((N, H), F32)

    r = lambda b: b.reshape(1, -1)
    enc_node = [(w, r(b)) for (w, b) in params["node_in"]]
    enc_edge = [(w, r(b)) for (w, b) in params["edge_in"]]
    dec = [(w, r(b)) for (w, b) in params["node_out"]]
    esplits = [_split_edge_params(l["lin_edge"], H) for l in params["in_layers"]]
    nsplits = [_split_node_params(l["lin_node"], H) for l in params["in_layers"]]

    batch_src = _sc_gather_batch_src(batch, src)
    counts = _tc_counts(batch_src[:, None], g)

    x, A, B = _tc_node_encode_prep(x4, enc_node, esplits[0][0], esplits[0][1])

    nlayers = len(params["in_layers"])
    ef = None
    for li in range(nlayers):
        ga, gb = _sc_gather_pair(A, B, dst, src)
        if li == 0:
            msg, ef = _tc_edge_encode_msg(attr, ga, gb, counts, gf,
                                          enc_edge, esplits[0][2])
        else:
            msg, ef = _tc_edge_msg(ef, ga, gb, counts, gf, esplits[li][2])
        partials = _sc_scatter_add(msg, dst, zeros_nh)
        if li + 1 < nlayers:
            x, A, B = _tc_node_update_prep(x, partials, batch2d, gf,
                                           nsplits[li], esplits[li + 1][0],
                                           esplits[li + 1][1])
        else:
            out = _tc_node_update_decode(x, partials, batch2d, gf,
                                         nsplits[li], dec)
    return out


# speedup vs baseline: 3.7752x; 1.1950x over previous
import functools

import jax
import jax.numpy as jnp
from jax import lax
from jax.experimental import pallas as pl
from jax.experimental.pallas import tpu as pltpu
from jax.experimental.pallas import tpu_sc as plsc

F32 = jnp.float32

NC = 2
NS = 16
NW = NC * NS

EBLK = 2000
NBLK = 2000
C = 128


def _dot(a, b):
    return lax.dot_general(a, b, (((1,), (0,)), ((), ())),
                           preferred_element_type=F32)


def _sc_gather_pair(A, B, dst, src):
    E = dst.shape[0]
    H = A.shape[1]
    nchunk = E // C
    niter = (nchunk + NW - 1) // NW
    npair = (niter + 1) // 2
    mesh = plsc.VectorSubcoreMesh(core_axis_name="c", subcore_axis_name="s")

    @functools.partial(
        pl.kernel, mesh=mesh,
        out_type=(jax.ShapeDtypeStruct((E, H), F32),
                  jax.ShapeDtypeStruct((E, H), F32)),
        scratch_types=[pltpu.VMEM((2, C), jnp.int32),
                       pltpu.VMEM((2, C), jnp.int32),
                       pltpu.VMEM((2, C, H), F32),
                       pltpu.VMEM((2, C, H), F32)]
                      + [pltpu.SemaphoreType.DMA] * 8,
    )
    def k(a_hbm, b_hbm, dst_hbm, src_hbm, ga_hbm, gb_hbm,
          idx_d, idx_s, buf_a, buf_b,
          sga0, sgb0, sga1, sgb1, swa0, swb0, swa1, swb1):
        wid = lax.axis_index("s") * NC + lax.axis_index("c")
        sg = ((sga0, sgb0), (sga1, sgb1))
        sw = ((swa0, swb0), (swa1, swb1))

        def half(kk, b, t):
            jj = wid + t * NW
            off = jj * C

            @pl.when(jnp.logical_and(jj < nchunk, kk > 0))
            def _():
                pltpu.make_async_copy(buf_a.at[b], ga_hbm.at[pl.ds(0, C)],
                                      sw[b][0]).wait()
                pltpu.make_async_copy(buf_b.at[b], gb_hbm.at[pl.ds(0, C)],
                                      sw[b][1]).wait()

            @pl.when(jj < nchunk)
            def _():
                pltpu.sync_copy(dst_hbm.at[pl.ds(off, C)], idx_d.at[b])
                pltpu.sync_copy(src_hbm.at[pl.ds(off, C)], idx_s.at[b])
                pltpu.async_copy(a_hbm.at[idx_d.at[b]], buf_a.at[b], sg[b][0])
                pltpu.async_copy(b_hbm.at[idx_s.at[b]], buf_b.at[b], sg[b][1])

        def drain(b, t):
            jj = wid + t * NW
            off = jj * C

            @pl.when(jj < nchunk)
            def _():
                pltpu.make_async_copy(a_hbm.at[idx_d.at[b]], buf_a.at[b],
                                      sg[b][0]).wait()
                pltpu.make_async_copy(b_hbm.at[idx_s.at[b]], buf_b.at[b],
                                      sg[b][1]).wait()
                pltpu.async_copy(buf_a.at[b], ga_hbm.at[pl.ds(off, C)],
                                 sw[b][0])
                pltpu.async_copy(buf_b.at[b], gb_hbm.at[pl.ds(off, C)],
                                 sw[b][1])

        def body(kk, carry):
            half(kk, 0, 2 * kk)
            half(kk, 1, 2 * kk + 1)
            drain(0, 2 * kk)
            drain(1, 2 * kk + 1)
            return carry

        lax.fori_loop(0, npair, body, 0)
        pltpu.make_async_copy(buf_a.at[0], ga_hbm.at[pl.ds(0, C)], sw[0][0]).wait()
        pltpu.make_async_copy(buf_b.at[0], gb_hbm.at[pl.ds(0, C)], sw[0][1]).wait()
        pltpu.make_async_copy(buf_a.at[1], ga_hbm.at[pl.ds(0, C)], sw[1][0]).wait()
        pltpu.make_async_copy(buf_b.at[1], gb_hbm.at[pl.ds(0, C)], sw[1][1]).wait()

    return k(A, B, dst, src)


def _sc_gather_batch_src(batch, src):
    E = src.shape[0]
    nchunk = E // C
    mesh = plsc.VectorSubcoreMesh(core_axis_name="c", subcore_axis_name="s")

    @functools.partial(
        pl.kernel, mesh=mesh,
        out_type=jax.ShapeDtypeStruct((E,), jnp.int32),
        scratch_types=[pltpu.VMEM((C,), jnp.int32),
                       pltpu.VMEM((C,), jnp.int32),
                       pltpu.SemaphoreType.DMA],
    )
    def k(batch_hbm, src_hbm, out_hbm, idx_v, buf, sem):
        wid = lax.axis_index("s") * NC + lax.axis_index("c")
        nj = (nchunk - wid + NW - 1) // NW

        def body(j, carry):
            jj = wid + j * NW
            off = jj * C
            pltpu.sync_copy(src_hbm.at[pl.ds(off, C)], idx_v)
            pltpu.async_copy(batch_hbm.at[idx_v], buf, sem).wait()
            pltpu.sync_copy(buf, out_hbm.at[pl.ds(off, C)])
            return carry

        lax.fori_loop(0, nj, body, 0)

    return k(batch, src)


def _sc_scatter_add(msg, dst, zeros_nh):
    E, H = msg.shape
    N = zeros_nh.shape[0]
    nchunk = E // C
    ZCH = 40
    nzc = N // ZCH
    mesh = plsc.VectorSubcoreMesh(core_axis_name="c", subcore_axis_name="s")

    niter = (nchunk + NW - 1) // NW
    npair = (niter + 1) // 2

    @functools.partial(
        pl.kernel, mesh=mesh,
        out_type=jax.ShapeDtypeStruct((NC, N, H), F32),
        scratch_types=[pltpu.VMEM((2, C), jnp.int32),
                       pltpu.VMEM((2, C, H), F32),
                       pltpu.VMEM_SHARED((N, H), F32)]
                      + [pltpu.SemaphoreType.DMA] * 4,
    )
    def k(msg_hbm, dst_hbm, zero_hbm, out_hbm, idx_v, buf, acc,
          sl0, sl1, ss0, ss1):
        cid = lax.axis_index("c")
        sid = lax.axis_index("s")
        wid = sid * NC + cid
        sl = (sl0, sl1)
        ss = (ss0, ss1)
        nz = (nzc - sid + NS - 1) // NS

        def zbody(j, carry):
            r0 = (sid + j * NS) * ZCH
            pltpu.sync_copy(zero_hbm.at[pl.ds(r0, ZCH)],
                            acc.at[pl.ds(r0, ZCH)])
            return carry

        lax.fori_loop(0, nz, zbody, 0)
        plsc.subcore_barrier()

        def phase1(kk, b, t):
            jj = wid + t * NW
            off = jj * C

            @pl.when(jnp.logical_and(jj < nchunk, kk > 0))
            def _():
                pltpu.make_async_copy(buf.at[b], acc.at[idx_v.at[b]],
                                      ss[b]).wait()

            @pl.when(jj < nchunk)
            def _():
                pltpu.sync_copy(dst_hbm.at[pl.ds(off, C)], idx_v.at[b])
                pltpu.async_copy(msg_hbm.at[pl.ds(off, C)], buf.at[b], sl[b])

        def phase2(b, t):
            jj = wid + t * NW
            off = jj * C

            @pl.when(jj < nchunk)
            def _():
                pltpu.make_async_copy(msg_hbm.at[pl.ds(off, C)], buf.at[b],
                                      sl[b]).wait()
                pltpu.async_copy(buf.at[b], acc.at[idx_v.at[b]], ss[b],
                                 add=True)

        def body(kk, carry):
            phase1(kk, 0, 2 * kk)
            phase1(kk, 1, 2 * kk + 1)
            phase2(0, 2 * kk)
            phase2(1, 2 * kk + 1)
            return carry

        lax.fori_loop(0, npair, body, 0)
        pltpu.make_async_copy(buf.at[0], acc.at[idx_v.at[0]], ss[0]).wait()
        pltpu.make_async_copy(buf.at[1], acc.at[idx_v.at[1]], ss[1]).wait()
        plsc.subcore_barrier()

        def wbody(j, carry):
            r0 = (sid + j * NS) * ZCH
            pltpu.sync_copy(acc.at[pl.ds(r0, ZCH)],
                            out_hbm.at[cid, pl.ds(r0, ZCH)])
            return carry

        lax.fori_loop(0, nz, wbody, 0)

    return k(msg, dst, zeros_nh)


def _tc_counts(batch_src_2d, g):
    E = batch_src_2d.shape[0]
    grid = (E // EBLK,)

    def body(bs_ref, out_ref):
        i = pl.program_id(0)

        @pl.when(i == 0)
        def _():
            out_ref[...] = jnp.zeros_like(out_ref)

        oh = (bs_ref[...] == lax.broadcasted_iota(jnp.int32, (EBLK, g), 1))
        out_ref[...] += jnp.sum(oh.astype(F32), axis=0, keepdims=True)

    return pl.pallas_call(
        body,
        grid=grid,
        in_specs=[pl.BlockSpec((EBLK, 1), lambda i: (i, 0))],
        out_specs=pl.BlockSpec((1, g), lambda i: (0, 0)),
        out_shape=jax.ShapeDtypeStruct((1, g), F32),
    )(batch_src_2d)


def _mlp3(h, w1, b1, w2, b2, w3, b3):
    h = jax.nn.relu(_dot(h, w1) + b1)
    h = jax.nn.relu(_dot(h, w2) + b2)
    return _dot(h, w3) + b3


def _tc_node_encode_prep(x4, enc, w1a_b1, w1b):
    N = x4.shape[0]
    H = enc[0][0].shape[1]
    grid = (N // NBLK,)
    (we1, be1), (we2, be2), (we3, be3) = enc
    w1a, b1 = w1a_b1

    def body(x4_ref, we1_r, be1_r, we2_r, be2_r, we3_r, be3_r,
             w1a_r, b1_r, w1b_r, x0_ref, a_ref, b_ref):
        x0 = _mlp3(x4_ref[...], we1_r[...], be1_r[...], we2_r[...],
                   be2_r[...], we3_r[...], be3_r[...])
        x0_ref[...] = x0
        a_ref[...] = _dot(x0, w1a_r[...]) + b1_r[...]
        b_ref[...] = _dot(x0, w1b_r[...])

    full = lambda a: pl.BlockSpec(a.shape, lambda i: (0,) * a.ndim)
    return pl.pallas_call(
        body,
        grid=grid,
        in_specs=[pl.BlockSpec((NBLK, 4), lambda i: (i, 0)),
                  full(we1), full(be1), full(we2), full(be2),
                  full(we3), full(be3), full(w1a), full(b1), full(w1b)],
        out_specs=[pl.BlockSpec((NBLK, H), lambda i: (i, 0))] * 3,
        out_shape=[jax.ShapeDtypeStruct((N, H), F32)] * 3,
    )(x4, we1, be1, we2, be2, we3, be3, w1a, b1, w1b)


def _edge_core(ga, gb, ef, e0, counts, gf, w1c, w1d, w2, b2, w3, b3, eblk, g):
    lt = (lax.broadcasted_iota(jnp.int32, (g, g), 0)
          < lax.broadcasted_iota(jnp.int32, (g, g), 1)).astype(F32)
    starts = _dot(counts, lt).astype(jnp.int32)
    ends = starts + counts.astype(jnp.int32)
    ridx = lax.broadcasted_iota(jnp.int32, (eblk, g), 0) + e0
    oh = (ridx >= starts).astype(F32) - (ridx >= ends).astype(F32)
    gf_tab = _dot(gf, w1d)
    h1 = jax.nn.relu(ga + gb + _dot(ef, w1c) + _dot(oh, gf_tab))
    h2 = jax.nn.relu(_dot(h1, w2) + b2)
    return _dot(h2, w3) + b3


def _tc_edge_encode_msg(attr, ga, gb, counts, gf, enc, esplit):
    E = attr.shape[0]
    H = ga.shape[1]
    g = gf.shape[0]
    grid = (E // EBLK,)
    (we1, be1), (we2, be2), (we3, be3) = enc
    w1c, w1d, w2, b2, w3, b3 = esplit

    def body(attr_ref, ga_ref, gb_ref, counts_r, gf_r,
             we1_r, be1_r, we2_r, be2_r, we3_r, be3_r,
             w1c_r, w1d_r, w2_r, b2_r, w3_r, b3_r,
             msg_ref, ef_ref):
        ef = _mlp3(attr_ref[...], we1_r[...], be1_r[...], we2_r[...],
                   be2_r[...], we3_r[...], be3_r[...])
        e0 = pl.program_id(0) * EBLK
        msg = _edge_core(ga_ref[...], gb_ref[...], ef, e0, counts_r[...],
                         gf_r[...], w1c_r[...], w1d_r[...], w2_r[...],
                         b2_r[...], w3_r[...], b3_r[...], EBLK, g)
        msg_ref[...] = msg
        ef_ref[...] = ef + msg

    full = lambda a: pl.BlockSpec(a.shape, lambda i: (0,) * a.ndim)
    return pl.pallas_call(
        body,
        grid=grid,
        in_specs=[pl.BlockSpec((EBLK, 2), lambda i: (i, 0)),
                  pl.BlockSpec((EBLK, H), lambda i: (i, 0)),
                  pl.BlockSpec((EBLK, H), lambda i: (i, 0)),
                  full(counts), full(gf),
                  full(we1), full(be1), full(we2), full(be2),
                  full(we3), full(be3),
                  full(w1c), full(w1d), full(w2), full(b2),
                  full(w3), full(b3)],
        out_specs=[pl.BlockSpec((EBLK, H), lambda i: (i, 0))] * 2,
        out_shape=[jax.ShapeDtypeStruct((E, H), F32)] * 2,
    )(attr, ga, gb, counts, gf, we1, be1, we2, be2, we3, be3,
      w1c, w1d, w2, b2, w3, b3)


def _tc_edge_msg(ef_in, ga, gb, counts, gf, esplit):
    E, H = ef_in.shape
    g = gf.shape[0]
    grid = (E // EBLK,)
    w1c, w1d, w2, b2, w3, b3 = esplit

    def body(ef_in_ref, ga_ref, gb_ref, counts_r, gf_r,
             w1c_r, w1d_r, w2_r, b2_r, w3_r, b3_r, msg_ref, ef_ref):
        ef = ef_in_ref[...]
        e0 = pl.program_id(0) * EBLK
        msg = _edge_core(ga_ref[...], gb_ref[...], ef, e0, counts_r[...],
                         gf_r[...], w1c_r[...], w1d_r[...], w2_r[...],
                         b2_r[...], w3_r[...], b3_r[...], EBLK, g)
        msg_ref[...] = msg
        ef_ref[...] = ef + msg

    full = lambda a: pl.BlockSpec(a.shape, lambda i: (0,) * a.ndim)
    return pl.pallas_call(
        body,
        grid=grid,
        in_specs=[pl.BlockSpec((EBLK, H), lambda i: (i, 0))] * 3
                 + [full(counts), full(gf), full(w1c), full(w1d),
                    full(w2), full(b2), full(w3), full(b3)],
        out_specs=[pl.BlockSpec((EBLK, H), lambda i: (i, 0))] * 2,
        out_shape=[jax.ShapeDtypeStruct((E, H), F32)] * 2,
    )(ef_in, ga, gb, counts, gf, w1c, w1d, w2, b2, w3, b3)


def _node_update(x, p_ref0, p_ref1, batch_blk, gf, u1a, u1b, u1c, c1,
                 u2, c2, u3, c3, g):
    aggr = p_ref0 + p_ref1
    oh = (batch_blk == lax.broadcasted_iota(jnp.int32, (batch_blk.shape[0], g), 1)).astype(F32)
    gfn_tab = _dot(gf, u1c)
    h1 = jax.nn.relu(_dot(x, u1a) + _dot(aggr, u1b) + _dot(oh, gfn_tab) + c1)
    h2 = jax.nn.relu(_dot(h1, u2) + c2)
    return x + _dot(h2, u3) + c3


def _tc_node_update_prep(x, partials, batch2d, gf, nsplit, w1a_b1, w1b):
    N, H = x.shape
    g = gf.shape[0]
    grid = (N // NBLK,)
    u1a, u1b, u1c, c1, u2, c2, u3, c3 = nsplit
    w1a, b1 = w1a_b1

    def body(x_ref, p_ref, b_ref_in, gf_r, u1a_r, u1b_r, u1c_r, c1_r,
             u2_r, c2_r, u3_r, c3_r, w1a_r, b1_r, w1b_r,
             xn_ref, a_ref, b_ref):
        xn = _node_update(x_ref[...], p_ref[0], p_ref[1], b_ref_in[...],
                          gf_r[...], u1a_r[...], u1b_r[...], u1c_r[...],
                          c1_r[...], u2_r[...], c2_r[...], u3_r[...],
                          c3_r[...], g)
        xn_ref[...] = xn
        a_ref[...] = _dot(xn, w1a_r[...]) + b1_r[...]
        b_ref[...] = _dot(xn, w1b_r[...])

    full = lambda a: pl.BlockSpec(a.shape, lambda i: (0,) * a.ndim)
    return pl.pallas_call(
        body,
        grid=grid,
        in_specs=[pl.BlockSpec((NBLK, H), lambda i: (i, 0)),
                  pl.BlockSpec((NC, NBLK, H), lambda i: (0, i, 0)),
                  pl.BlockSpec((NBLK, 1), lambda i: (i, 0)),
                  full(gf), full(u1a), full(u1b), full(u1c), full(c1),
                  full(u2), full(c2), full(u3), full(c3),
                  full(w1a), full(b1), full(w1b)],
        out_specs=[pl.BlockSpec((NBLK, H), lambda i: (i, 0))] * 3,
        out_shape=[jax.ShapeDtypeStruct((N, H), F32)] * 3,
    )(x, partials, batch2d, gf, u1a, u1b, u1c, c1, u2, c2, u3, c3,
      w1a, b1, w1b)


def _tc_node_update_decode(x, partials, batch2d, gf, nsplit, dec):
    N, H = x.shape
    g = gf.shape[0]
    grid = (N // NBLK,)
    u1a, u1b, u1c, c1, u2, c2, u3, c3 = nsplit
    (wd1, bd1), (wd2, bd2), (wd3, bd3) = dec
    O = wd3.shape[1]

    def body(x_ref, p_ref, b_ref_in, gf_r, u1a_r, u1b_r, u1c_r, c1_r,
             u2_r, c2_r, u3_r, c3_r, wd1_r, bd1_r, wd2_r, bd2_r,
             wd3_r, bd3_r, out_ref):
        xn = _node_update(x_ref[...], p_ref[0], p_ref[1], b_ref_in[...],
                          gf_r[...], u1a_r[...], u1b_r[...], u1c_r[...],
                          c1_r[...], u2_r[...], c2_r[...], u3_r[...],
                          c3_r[...], g)
        out_ref[...] = _mlp3(xn, wd1_r[...], bd1_r[...], wd2_r[...],
                             bd2_r[...], wd3_r[...], bd3_r[...])

    full = lambda a: pl.BlockSpec(a.shape, lambda i: (0,) * a.ndim)
    return pl.pallas_call(
        body,
        grid=grid,
        in_specs=[pl.BlockSpec((NBLK, H), lambda i: (i, 0)),
                  pl.BlockSpec((NC, NBLK, H), lambda i: (0, i, 0)),
                  pl.BlockSpec((NBLK, 1), lambda i: (i, 0)),
                  full(gf), full(u1a), full(u1b), full(u1c), full(c1),
                  full(u2), full(c2), full(u3), full(c3),
                  full(wd1), full(bd1), full(wd2), full(bd2),
                  full(wd3), full(bd3)],
        out_specs=pl.BlockSpec((NBLK, O), lambda i: (i, 0)),
        out_shape=jax.ShapeDtypeStruct((N, O), F32),
    )(x, partials, batch2d, gf, u1a, u1b, u1c, c1, u2, c2, u3, c3,
      wd1, bd1, wd2, bd2, wd3, bd3)


def _split_edge_params(lin_edge, h):
    (w1, b1), (w2, b2), (w3, b3) = lin_edge
    w1a = w1[:h]
    w1b = w1[h:2 * h]
    w1c = w1[2 * h:3 * h]
    w1d = w1[3 * h:]
    r = lambda b: b.reshape(1, -1)
    return (w1a, r(b1)), w1b, (w1c, w1d, w2, r(b2), w3, r(b3))


def _split_node_params(lin_node, h):
    (u1, c1), (u2, c2), (u3, c3) = lin_node
    r = lambda b: b.reshape(1, -1)
    return (u1[:h], u1[h:2 * h], u1[2 * h:], r(c1), u2, r(c2), u3, r(c3))


def kernel(initial_position, contact_node, parent2child, branch,
           contact_force, edge_index, batch, params):
    N = initial_position.shape[0]
    E = parent2child.shape[0]
    H = params["node_in"][-1][0].shape[1]
    gf = contact_force.reshape(-1, 3)
    g = gf.shape[0]

    src = edge_index[0]
    dst = edge_index[1]
    x4 = jnp.concatenate([initial_position, contact_node[:, None]], axis=-1)
    attr = jnp.stack([parent2child, branch], axis=-1)
    batch2d = batch[:, None]
    zeros_nh = jnp.zeros((N, H), F32)

    r = lambda b: b.reshape(1, -1)
    enc_node = [(w, r(b)) for (w, b) in params["node_in"]]
    enc_edge = [(w, r(b)) for (w, b) in params["edge_in"]]
    dec = [(w, r(b)) for (w, b) in params["node_out"]]
    esplits = [_split_edge_params(l["lin_edge"], H) for l in params["in_layers"]]
    nsplits = [_split_node_params(l["lin_node"], H) for l in params["in_layers"]]

    batch_src = _sc_gather_batch_src(batch, src)
    counts = _tc_counts(batch_src[:, None], g)

    x, A, B = _tc_node_encode_prep(x4, enc_node, esplits[0][0], esplits[0][1])

    nlayers = len(params["in_layers"])
    ef = None
    for li in range(nlayers):
        ga, gb = _sc_gather_pair(A, B, dst, src)
        if li == 0:
            msg, ef = _tc_edge_encode_msg(attr, ga, gb, counts, gf,
                                          enc_edge, esplits[0][2])
        else:
            msg, ef = _tc_edge_msg(ef, ga, gb, counts, gf, esplits[li][2])
        partials = _sc_scatter_add(msg, dst, zeros_nh)
        if li + 1 < nlayers:
            x, A, B = _tc_node_update_prep(x, partials, batch2d, gf,
                                           nsplits[li], esplits[li + 1][0],
                                           esplits[li + 1][1])
        else:
            out = _tc_node_update_decode(x, partials, batch2d, gf,
                                         nsplits[li], dec)
    return out
